# baseline (speedup 1.0000x reference)
import functools

import jax
import jax.numpy as jnp
from jax import lax
from jax.experimental import pallas as pl
from jax.experimental.pallas import tpu as pltpu
from jax.experimental.pallas import tpu_sc as plsc

NC = 2
NS = 16
L = 16
NW = NC * NS
B = 16384
D = 32
SR = 128
RPS = SR // D
NUM_ROWS = 1000000
HOFF = NUM_ROWS // RPS
BPW = B // NW
CHUNK = 128
NCHUNK = BPW // CHUNK
GPC = CHUNK // L

_LOG1P = (
    3.910905549409094e-08, 0.9999936302585134, -0.4998254986434647,
    0.33144665224336606, -0.2394333707458602, 0.16499812983396112,
    -0.09229041738050231, 0.03426459995555095, -0.006006605050865348,
)


def _log1p_poly(t):
    acc = jnp.full_like(t, _LOG1P[-1])
    for c in reversed(_LOG1P[:-1]):
        acc = acc * t + jnp.float32(c)
    return acc


@functools.cache
def _build_bpr_sc():
  mesh = plsc.VectorSubcoreMesh(
      core_axis_name="c", subcore_axis_name="s", num_cores=NC, num_subcores=NS)

  @functools.partial(
      pl.kernel,
      out_type=jax.ShapeDtypeStruct((NW, L), jnp.float32),
      mesh=mesh,
      scratch_types=[
          pltpu.VMEM((BPW,), jnp.int32),
          pltpu.VMEM((BPW,), jnp.int32),
          pltpu.VMEM((BPW,), jnp.int32),
          pltpu.VMEM((BPW,), jnp.int32),
          pltpu.VMEM((BPW,), jnp.int32),
          pltpu.VMEM((BPW,), jnp.int32),
          [pltpu.VMEM((CHUNK, SR), jnp.float32)] * 6,
          pltpu.VMEM((L,), jnp.float32),
          [pltpu.SemaphoreType.DMA] * 2,
      ],
      compiler_params=pltpu.CompilerParams(needs_layout_passes=False),
  )
  def _bpr_sc(us_hbm, is_hbm, js_hbm, uo_hbm, io_hbm, jo_hbm, wh_hbm, out_hbm,
              us_v, is_v, js_v, uo_v, io_v, jo_v, bufs, o_v, sems):
    wid = lax.axis_index("s") * NC + lax.axis_index("c")

    pltpu.sync_copy(us_hbm.at[wid], us_v)
    pltpu.sync_copy(is_hbm.at[wid], is_v)
    pltpu.sync_copy(js_hbm.at[wid], js_v)
    pltpu.sync_copy(uo_hbm.at[wid], uo_v)
    pltpu.sync_copy(io_hbm.at[wid], io_v)
    pltpu.sync_copy(jo_hbm.at[wid], jo_v)

    iota = lax.iota(jnp.int32, L)
    zero = jnp.zeros((L,), jnp.float32)

    def issue_chunk(c, par):
      ue_v, ie_v, je_v = bufs[par * 3:par * 3 + 3]
      sem = sems[par]
      rows = pl.ds(c * CHUNK, CHUNK)
      return [pltpu.async_copy(wh_hbm.at[us_v.at[rows]], ue_v, sem),
              pltpu.async_copy(wh_hbm.at[is_v.at[rows]], ie_v, sem),
              pltpu.async_copy(wh_hbm.at[js_v.at[rows]], je_v, sem)]

    def compute_chunk(c, par, acc):
      ue_v, ie_v, je_v = bufs[par * 3:par * 3 + 3]

      def body(g, a, c=c):
        base = c * CHUNK + g * L
        r_idx = g * L + iota
        cu = uo_v[pl.ds(base, L)]
        ci = io_v[pl.ds(base, L)]
        cj = jo_v[pl.ds(base, L)]
        x = zero
        for d in range(D):
          ue = plsc.load_gather(ue_v, [r_idx, cu + d])
          ie = plsc.load_gather(ie_v, [r_idx, ci + d])
          je = plsc.load_gather(je_v, [r_idx, cj + d])
          x = x + ue * (ie - je)
        t = jnp.exp(-jnp.abs(x))
        return a + jnp.minimum(x, 0.0) - _log1p_poly(t)

      return lax.fori_loop(0, GPC, body, acc)

    acc = zero
    pending = issue_chunk(0, 0)
    for c in range(NCHUNK):
      nxt = issue_chunk(c + 1, (c + 1) % 2) if c + 1 < NCHUNK else []
      for cp in pending:
        cp.wait()
      pending = nxt
      acc = compute_chunk(c, c % 2, acc)

    o_v[...] = acc
    pltpu.sync_copy(o_v, out_hbm.at[wid])

  return _bpr_sc


def kernel(u, i, j, W, H):
    u = u.astype(jnp.int32)
    i = i.astype(jnp.int32)
    j = j.astype(jnp.int32)
    us = (u // RPS).reshape(NW, BPW)
    is_ = (i // RPS + HOFF).reshape(NW, BPW)
    js = (j // RPS + HOFF).reshape(NW, BPW)
    uo = (u % RPS * D).reshape(NW, BPW)
    io = (i % RPS * D).reshape(NW, BPW)
    jo = (j % RPS * D).reshape(NW, BPW)
    wh = jnp.concatenate([W.reshape(-1, SR), H.reshape(-1, SR)])
    partials = _build_bpr_sc()(us, is_, js, uo, io, jo, wh)
    return -jnp.sum(partials)
